# Initial kernel scaffold; baseline (speedup 1.0000x reference)
import jax
import jax.numpy as jnp
from jax.experimental import pallas as pl


def kernel(ids, embedding_var, default_embedding):
    raise NotImplementedError("write your pallas kernel here")


# speedup vs baseline: 90.9820x; 90.9820x over previous
import functools

import jax
import jax.numpy as jnp
from jax import lax
from jax.experimental import pallas as pl
from jax.experimental.pallas import tpu as pltpu
from jax.experimental.pallas import tpu_sc as plsc

N = 1024 * 26 * 20
DIM = 32
VOCAB = 1000000
L = 16
NC, NS = 2, 16
NW = NC * NS
CH = N // NW
ROWS = N // 128
CR = ROWS // NW
VP = 1000448
SH = VP // NW
SENT = 2**30

_mesh = plsc.VectorSubcoreMesh(
    core_axis_name="c", subcore_axis_name="s", num_cores=NC, num_subcores=NS)
_params = pltpu.CompilerParams(
    needs_layout_passes=False, use_tc_tiling_on_sc=False)


def _wid():
  return lax.axis_index("s") * NC + lax.axis_index("c")


def _iota16():
  return lax.iota(jnp.int32, 16)


def _ka(ids_hbm, fpt_hbm, tbl, ib0, ib1, s0, s1):
  wid = _wid()
  base = wid * SH
  hi = base + SH

  sent = jnp.full((L,), SENT, jnp.int32)

  def init_body(i, _):
    i = i.astype(jnp.int32)
    tbl[pl.ds(i * L, L)] = sent
    return 0

  lax.fori_loop(0, SH // L, init_body, 0, unroll=4)

  revi = 15 - _iota16()

  def start(j, buf, sem):
    pltpu.make_async_copy(ids_hbm.at[pl.ds(j * 32, 32), :], buf, sem).start()

  def wait(buf, sem):
    pltpu.make_async_copy(ids_hbm.at[pl.ds(0, 32), :], buf, sem).wait()

  def process(buf, pbase):
    def row(r, _):
      r = r.astype(jnp.int32)
      for c in range(8):
        v = buf[r, pl.ds(c * L, L)]
        rid = lax.rev(v, (0,))
        m = (rid >= base) & (rid < hi)
        rpos = (pbase + r * 128 + c * L) + revi
        _, lastm = plsc.scan_count(rid, mask=m)
        lm = lastm & m
        idx = jnp.where(m, rid - base, 0)
        cur = plsc.load_gather(tbl, [idx], mask=lm)
        plsc.store_scatter(tbl, [idx], jnp.minimum(cur, rpos), mask=lm)
      return 0

    lax.fori_loop(0, 32, row, 0)

  start(0, ib0, s0)

  def piece(g, _):
    j = 2 * g.astype(jnp.int32)
    start(j + 1, ib1, s1)
    wait(ib0, s0)
    process(ib0, j * 4096)

    @pl.when(j + 2 < ROWS // 32)
    def _():
      start(j + 2, ib0, s0)

    wait(ib1, s1)
    process(ib1, (j + 1) * 4096)
    return 0

  lax.fori_loop(0, ROWS // 64, piece, 0)

  pltpu.sync_copy(tbl, fpt_hbm.at[pl.ds(base, SH)])


_phase_a = functools.partial(
    pl.kernel,
    out_type=jax.ShapeDtypeStruct((VP,), jnp.int32),
    mesh=_mesh,
    compiler_params=_params,
    scratch_types=[
        pltpu.VMEM((SH,), jnp.int32),
        pltpu.VMEM((32, 128), jnp.int32),
        pltpu.VMEM((32, 128), jnp.int32),
        pltpu.SemaphoreType.DMA,
        pltpu.SemaphoreType.DMA,
    ],
)(_ka)


def _kb(ids_hbm, fpt_hbm, fpo_hbm, s_hbm, part_hbm, idv, fpv, pbuf, sem):
  wid = _wid()
  rbase = wid * CR

  pltpu.sync_copy(ids_hbm.at[pl.ds(rbase, CR), :], idv)

  def ggrp(g, _):
    g = g.astype(jnp.int32)
    for b in range(10):
      j = g * 10 + b
      pltpu.make_async_copy(fpt_hbm.at[idv.at[j]], fpv.at[j], sem).start()
    for b in range(10):
      pltpu.make_async_copy(fpt_hbm.at[idv.at[0]], fpv.at[0], sem).wait()
    return 0

  lax.fori_loop(0, CR // 10, ggrp, 0)

  iot = _iota16()
  pbase = rbase * 128

  def crow(r, carry):
    r = r.astype(jnp.int32)
    for c in range(8):
      v = fpv[r, pl.ds(c * L, L)]
      pos = (pbase + r * 128 + c * L) + iot
      isf = jnp.where(v == pos, 1, 0).astype(jnp.int32)
      cs = plsc.cumsum(isf)
      idv[r, pl.ds(c * L, L)] = cs - isf + carry
      carry = carry + jnp.sum(isf, dtype=jnp.int32)
    return carry

  total = lax.fori_loop(0, CR, crow, jnp.int32(0))

  pltpu.sync_copy(idv, s_hbm.at[pl.ds(rbase, CR), :])
  pltpu.sync_copy(fpv, fpo_hbm.at[pl.ds(rbase, CR), :])
  pbuf[...] = jnp.full((L,), 0, jnp.int32) + total
  pltpu.sync_copy(pbuf, part_hbm.at[wid])


_phase_b = functools.partial(
    pl.kernel,
    out_type=(
        jax.ShapeDtypeStruct((ROWS, 128), jnp.int32),
        jax.ShapeDtypeStruct((ROWS, 128), jnp.int32),
        jax.ShapeDtypeStruct((NW, L), jnp.int32),
    ),
    mesh=_mesh,
    compiler_params=_params,
    scratch_types=[
        pltpu.VMEM((CR, 128), jnp.int32),
        pltpu.VMEM((CR, 128), jnp.int32),
        pltpu.VMEM((L,), jnp.int32),
        pltpu.SemaphoreType.DMA,
    ],
)(_kb)


def _kc(fpo_hbm, s_hbm, part_hbm, emb_hbm, out_hbm,
        pv, offv, fpv, sv, rb0, rb1, sem, sr0, sr1):
  wid = _wid()
  rbase = wid * CR

  pltpu.sync_copy(part_hbm, pv)
  iot = _iota16()
  z = jnp.zeros((L,), jnp.int32)
  t0 = plsc.load_gather(pv, [iot, z])
  t1 = plsc.load_gather(pv, [iot + 16, z])
  c0 = plsc.cumsum(t0)
  c1 = plsc.cumsum(t1) + jnp.sum(t0, dtype=jnp.int32)
  offv[pl.ds(0, L)] = c0 - t0
  offv[pl.ds(L, L)] = c1 - t1

  pltpu.sync_copy(fpo_hbm.at[pl.ds(rbase, CR), :], fpv)

  def ggrp(g, _):
    g = g.astype(jnp.int32)
    for b in range(10):
      j = g * 10 + b
      pltpu.make_async_copy(s_hbm.at[fpv.at[j]], sv.at[j], sem).start()
    for b in range(10):
      pltpu.make_async_copy(s_hbm.at[fpv.at[0]], sv.at[0], sem).wait()
    return 0

  lax.fori_loop(0, CR // 10, ggrp, 0)

  def hrow(r, _):
    r = r.astype(jnp.int32)
    for c in range(8):
      f = fpv[r, pl.ds(c * L, L)]
      s = sv[r, pl.ds(c * L, L)]
      o = plsc.load_gather(offv, [f // CH])
      sv[r, pl.ds(c * L, L)] = s + o
    return 0

  lax.fori_loop(0, CR, hrow, 0)

  obase = wid * CH

  def gst(j, buf, sem_):
    pltpu.make_async_copy(emb_hbm.at[sv.at[j]], buf, sem_).start()

  def gwt(buf, sem_):
    pltpu.make_async_copy(emb_hbm.at[sv.at[0]], buf, sem_).wait()

  gst(0, rb0, sr0)

  def grow(g, _):
    j = 2 * g.astype(jnp.int32)
    gst(j + 1, rb1, sr1)
    gwt(rb0, sr0)
    pltpu.sync_copy(rb0, out_hbm.at[pl.ds(obase + j * 128, 128), :])

    @pl.when(j + 2 < CR)
    def _():
      gst(j + 2, rb0, sr0)

    gwt(rb1, sr1)
    pltpu.sync_copy(rb1, out_hbm.at[pl.ds(obase + (j + 1) * 128, 128), :])
    return 0

  lax.fori_loop(0, CR // 2, grow, 0)


_phase_c = functools.partial(
    pl.kernel,
    out_type=jax.ShapeDtypeStruct((N, DIM), jnp.float32),
    mesh=_mesh,
    compiler_params=_params,
    scratch_types=[
        pltpu.VMEM((NW, L), jnp.int32),
        pltpu.VMEM((NW,), jnp.int32),
        pltpu.VMEM((CR, 128), jnp.int32),
        pltpu.VMEM((CR, 128), jnp.int32),
        pltpu.VMEM((128, DIM), jnp.float32),
        pltpu.VMEM((128, DIM), jnp.float32),
        pltpu.SemaphoreType.DMA,
        pltpu.SemaphoreType.DMA,
        pltpu.SemaphoreType.DMA,
    ],
)(_kc)


def kernel(ids, embedding_var, default_embedding):
  del default_embedding
  ids32 = ids.reshape(-1).astype(jnp.int32).reshape(ROWS, 128)
  with jax.enable_x64(False):
    fpt = _phase_a(ids32)
    fpo, s_local, partials = _phase_b(ids32, fpt)
    out = _phase_c(fpo, s_local.reshape(-1), partials, embedding_var)
  return out.reshape(ids.shape + (DIM,))


# speedup vs baseline: 112.5039x; 1.2366x over previous
import functools

import jax
import jax.numpy as jnp
from jax import lax
from jax.experimental import pallas as pl
from jax.experimental.pallas import tpu as pltpu
from jax.experimental.pallas import tpu_sc as plsc

N = 1024 * 26 * 20
DIM = 32
VOCAB = 1000000
L = 16
NC, NS = 2, 16
NW = NC * NS
CH = N // NW
ROWS = N // 128
CR = ROWS // NW
HR = ROWS // NC
N2 = N // NC
NP = HR // 32
VP = 1000448
SHH = VP // NS
SENT = 2**30

_mesh = plsc.VectorSubcoreMesh(
    core_axis_name="c", subcore_axis_name="s", num_cores=NC, num_subcores=NS)
_params = pltpu.CompilerParams(
    needs_layout_passes=False, use_tc_tiling_on_sc=False)


def _wid():
  return lax.axis_index("s") * NC + lax.axis_index("c")


def _iota16():
  return lax.iota(jnp.int32, 16)


def _ka(idp_hbm, ta_hbm, tb_hbm, ids32_hbm, tbl, cbuf, ib0, ib1, s0, s1):
  sid = lax.axis_index("s")
  cid = lax.axis_index("c")
  base = sid * SHH
  hi = base + SHH

  sent = jnp.full((L,), SENT, jnp.int32)

  def init_body(i, _):
    i = i.astype(jnp.int32)
    tbl[pl.ds(i * L, L)] = sent
    return 0

  lax.fori_loop(0, SHH // L, init_body, 0, unroll=4)

  iot = _iota16()
  row0 = cid * HR

  def start(j, buf, sem):
    pltpu.make_async_copy(
        idp_hbm.at[pl.ds(row0 + j * 32, 32), :], buf, sem).start()

  def wait(buf, sem):
    pltpu.make_async_copy(idp_hbm.at[pl.ds(0, 32), :], buf, sem).wait()

  def process(buf, j):
    grow = row0 + j * 32 + sid * 2
    for rr in range(2):
      rv = jnp.full((L,), sid * 2 + rr, jnp.int32)
      for c in range(8):
        v = plsc.load_gather(buf, [rv, 32 * c + 2 * iot])
        cbuf[rr, pl.ds(c * L, L)] = v
    pltpu.sync_copy(cbuf, ids32_hbm.at[pl.ds(grow, 2), :])

    pbase = cid * N2 + j * 4096

    def row(rr, _):
      r = 31 - rr.astype(jnp.int32)
      rv = jnp.full((L,), r, jnp.int32)
      for c in range(7, -1, -1):
        rid = plsc.load_gather(buf, [rv, 32 * c + 30 - 2 * iot])
        m = (rid >= base) & (rid < hi)
        _, lastm = plsc.scan_count(rid, mask=m)
        lm = lastm & m
        idx = jnp.where(m, rid - base, 0)
        rpos = (pbase + r * 128 + c * L + 15) - iot
        plsc.store_scatter(tbl, [idx], rpos, mask=lm)
      return 0

    lax.fori_loop(0, 32, row, 0)

  start(NP - 1, ib0, s0)

  def piece(g, _):
    g = g.astype(jnp.int32)
    j0 = (NP - 1) - 2 * g

    @pl.when(j0 >= 1)
    def _():
      start(j0 - 1, ib1, s1)

    wait(ib0, s0)
    process(ib0, j0)

    j1 = j0 - 1

    @pl.when(j1 >= 0)
    def _():
      @pl.when(j1 >= 1)
      def _():
        start(j1 - 1, ib0, s0)

      wait(ib1, s1)
      process(ib1, j1)

    return 0

  lax.fori_loop(0, (NP + 1) // 2, piece, 0)

  @pl.when(cid == 0)
  def _():
    pltpu.sync_copy(tbl, ta_hbm.at[pl.ds(base, SHH)])

  @pl.when(cid == 1)
  def _():
    pltpu.sync_copy(tbl, tb_hbm.at[pl.ds(base, SHH)])


_phase_a = functools.partial(
    pl.kernel,
    out_type=(
        jax.ShapeDtypeStruct((VP,), jnp.int32),
        jax.ShapeDtypeStruct((VP,), jnp.int32),
        jax.ShapeDtypeStruct((ROWS, 128), jnp.int32),
    ),
    mesh=_mesh,
    compiler_params=_params,
    scratch_types=[
        pltpu.VMEM((SHH,), jnp.int32),
        pltpu.VMEM((2, 128), jnp.int32),
        pltpu.VMEM((32, 256), jnp.int32),
        pltpu.VMEM((32, 256), jnp.int32),
        pltpu.SemaphoreType.DMA,
        pltpu.SemaphoreType.DMA,
    ],
)(_ka)


def _kb(ids_hbm, ta_hbm, tb_hbm, fpo_hbm, s_hbm, part_hbm,
        idv, fpva, fpvb, sbuf, pbuf, sem):
  wid = _wid()
  rbase = wid * CR
  pbase = rbase * 128

  pltpu.sync_copy(ids_hbm.at[pl.ds(rbase, CR), :], idv)

  def mkgrp(src, dst):
    def ggrp(g, _):
      g = g.astype(jnp.int32)
      for b in range(10):
        j = g * 10 + b
        pltpu.make_async_copy(src.at[idv.at[j]], dst.at[j], sem).start()
      for b in range(10):
        pltpu.make_async_copy(src.at[idv.at[0]], dst.at[0], sem).wait()
      return 0
    return ggrp

  lax.fori_loop(0, CR // 10, mkgrp(ta_hbm, fpva), 0)
  lax.fori_loop(0, CR // 10, mkgrp(tb_hbm, fpvb), 0)

  iot = _iota16()

  def crow(r, carry):
    r = r.astype(jnp.int32)
    for c in range(8):
      v = jnp.minimum(fpva[r, pl.ds(c * L, L)], fpvb[r, pl.ds(c * L, L)])
      fpva[r, pl.ds(c * L, L)] = v
      pos = (pbase + r * 128 + c * L) + iot
      isf = jnp.where(v == pos, 1, 0).astype(jnp.int32)
      cs = plsc.cumsum(isf)
      sbuf[pl.ds(r * 128 + c * L, L)] = cs - isf + carry
      carry = carry + jnp.sum(isf, dtype=jnp.int32)
    return carry

  total = lax.fori_loop(0, CR, crow, jnp.int32(0))

  pltpu.sync_copy(fpva, fpo_hbm.at[pl.ds(rbase, CR), :])
  pltpu.sync_copy(sbuf, s_hbm.at[pl.ds(pbase, CH)])
  pbuf[...] = jnp.full((L,), 0, jnp.int32) + total
  pltpu.sync_copy(pbuf, part_hbm.at[wid])


_phase_b = functools.partial(
    pl.kernel,
    out_type=(
        jax.ShapeDtypeStruct((ROWS, 128), jnp.int32),
        jax.ShapeDtypeStruct((N,), jnp.int32),
        jax.ShapeDtypeStruct((NW, L), jnp.int32),
    ),
    mesh=_mesh,
    compiler_params=_params,
    scratch_types=[
        pltpu.VMEM((CR, 128), jnp.int32),
        pltpu.VMEM((CR, 128), jnp.int32),
        pltpu.VMEM((CR, 128), jnp.int32),
        pltpu.VMEM((CH,), jnp.int32),
        pltpu.VMEM((L,), jnp.int32),
        pltpu.SemaphoreType.DMA,
    ],
)(_kb)


def _kc(fpo_hbm, s_hbm, part_hbm, emb_hbm, out_hbm,
        pv, offv, fpv, sv, rb0, rb1, sem, sr0, sr1):
  wid = _wid()
  rbase = wid * CR

  pltpu.sync_copy(part_hbm, pv)
  iot = _iota16()
  z = jnp.zeros((L,), jnp.int32)
  t0 = plsc.load_gather(pv, [iot, z])
  t1 = plsc.load_gather(pv, [iot + 16, z])
  c0 = plsc.cumsum(t0)
  c1 = plsc.cumsum(t1) + jnp.sum(t0, dtype=jnp.int32)
  offv[pl.ds(0, L)] = c0 - t0
  offv[pl.ds(L, L)] = c1 - t1

  pltpu.sync_copy(fpo_hbm.at[pl.ds(rbase, CR), :], fpv)

  def ggrp(g, _):
    g = g.astype(jnp.int32)
    for b in range(10):
      j = g * 10 + b
      pltpu.make_async_copy(s_hbm.at[fpv.at[j]], sv.at[j], sem).start()
    for b in range(10):
      pltpu.make_async_copy(s_hbm.at[fpv.at[0]], sv.at[0], sem).wait()
    return 0

  lax.fori_loop(0, CR // 10, ggrp, 0)

  def hrow(r, _):
    r = r.astype(jnp.int32)
    for c in range(8):
      f = fpv[r, pl.ds(c * L, L)]
      s = sv[r, pl.ds(c * L, L)]
      o = plsc.load_gather(offv, [f // CH])
      sv[r, pl.ds(c * L, L)] = s + o
    return 0

  lax.fori_loop(0, CR, hrow, 0)

  obase = wid * CH

  def gst(j, buf, sem_):
    pltpu.make_async_copy(emb_hbm.at[sv.at[j]], buf, sem_).start()

  def gwt(buf, sem_):
    pltpu.make_async_copy(emb_hbm.at[sv.at[0]], buf, sem_).wait()

  gst(0, rb0, sr0)

  def grow(g, _):
    j = 2 * g.astype(jnp.int32)
    gst(j + 1, rb1, sr1)
    gwt(rb0, sr0)
    pltpu.sync_copy(rb0, out_hbm.at[pl.ds(obase + j * 128, 128), :])

    @pl.when(j + 2 < CR)
    def _():
      gst(j + 2, rb0, sr0)

    gwt(rb1, sr1)
    pltpu.sync_copy(rb1, out_hbm.at[pl.ds(obase + (j + 1) * 128, 128), :])
    return 0

  lax.fori_loop(0, CR // 2, grow, 0)


_phase_c = functools.partial(
    pl.kernel,
    out_type=jax.ShapeDtypeStruct((N, DIM), jnp.float32),
    mesh=_mesh,
    compiler_params=_params,
    scratch_types=[
        pltpu.VMEM((NW, L), jnp.int32),
        pltpu.VMEM((NW,), jnp.int32),
        pltpu.VMEM((CR, 128), jnp.int32),
        pltpu.VMEM((CR, 128), jnp.int32),
        pltpu.VMEM((128, DIM), jnp.float32),
        pltpu.VMEM((128, DIM), jnp.float32),
        pltpu.SemaphoreType.DMA,
        pltpu.SemaphoreType.DMA,
        pltpu.SemaphoreType.DMA,
    ],
)(_kc)


def kernel(ids, embedding_var, default_embedding):
  del default_embedding
  idp = lax.bitcast_convert_type(ids.reshape(-1), jnp.int32).reshape(ROWS, 256)
  with jax.enable_x64(False):
    ta, tb, ids32 = _phase_a(idp)
    fpo, s_local, partials = _phase_b(ids32, ta, tb)
    out = _phase_c(fpo, s_local, partials, embedding_var)
  return out.reshape(ids.shape + (DIM,))


# speedup vs baseline: 147.0601x; 1.3072x over previous
import functools

import jax
import jax.numpy as jnp
from jax import lax
from jax.experimental import pallas as pl
from jax.experimental.pallas import tpu as pltpu
from jax.experimental.pallas import tpu_sc as plsc

N = 1024 * 26 * 20
DIM = 32
VOCAB = 1000000
L = 16
NC, NS = 2, 16
NW = NC * NS
CH = N // NW
ROWS = N // 128
CR = ROWS // NW
HR = ROWS // NC
N2 = N // NC
NP = HR // 32
VP = 1000448
SHH = VP // NS
SENT = 2**30

_mesh = plsc.VectorSubcoreMesh(
    core_axis_name="c", subcore_axis_name="s", num_cores=NC, num_subcores=NS)
_params = pltpu.CompilerParams(
    needs_layout_passes=False, use_tc_tiling_on_sc=False)


def _wid():
  return lax.axis_index("s") * NC + lax.axis_index("c")


def _iota16():
  return lax.iota(jnp.int32, 16)


def _ka(idr_hbm, ta_hbm, tb_hbm, tbl, ib0, ib1, s0, s1):
  sid = lax.axis_index("s")
  cid = lax.axis_index("c")
  base = sid * SHH
  hi = base + SHH

  sent = jnp.full((L,), SENT, jnp.int32)

  def init_body(i, _):
    i = i.astype(jnp.int32)
    tbl[pl.ds(i * L, L)] = sent
    return 0

  lax.fori_loop(0, SHH // L, init_body, 0, unroll=4)

  iot = _iota16()
  row0 = cid * HR

  def start(j, buf, sem):
    pltpu.make_async_copy(
        idr_hbm.at[pl.ds(row0 + j * 32, 32), :], buf, sem).start()

  def wait(buf, sem):
    pltpu.make_async_copy(idr_hbm.at[pl.ds(0, 32), :], buf, sem).wait()

  def process(buf, j):
    pbase = cid * N2 + j * 4096

    def row(rr, _):
      r = 31 - rr.astype(jnp.int32)
      for c in range(7, -1, -1):
        rid = lax.rev(buf[r, pl.ds(c * L, L)], (0,))
        m = (rid >= base) & (rid < hi)
        _, lastm = plsc.scan_count(rid, mask=m)
        lm = lastm & m
        idx = jnp.where(m, rid - base, 0)
        rpos = (pbase + r * 128 + c * L + 15) - iot
        plsc.store_scatter(tbl, [idx], rpos, mask=lm)
      return 0

    lax.fori_loop(0, 32, row, 0)

  start(NP - 1, ib0, s0)

  def piece(g, _):
    g = g.astype(jnp.int32)
    j0 = (NP - 1) - 2 * g

    @pl.when(j0 >= 1)
    def _():
      start(j0 - 1, ib1, s1)

    wait(ib0, s0)
    process(ib0, j0)

    j1 = j0 - 1

    @pl.when(j1 >= 0)
    def _():
      @pl.when(j1 >= 1)
      def _():
        start(j1 - 1, ib0, s0)

      wait(ib1, s1)
      process(ib1, j1)

    return 0

  lax.fori_loop(0, (NP + 1) // 2, piece, 0)

  @pl.when(cid == 0)
  def _():
    pltpu.sync_copy(tbl, ta_hbm.at[pl.ds(base, SHH)])

  @pl.when(cid == 1)
  def _():
    pltpu.sync_copy(tbl, tb_hbm.at[pl.ds(base, SHH)])


_phase_a = functools.partial(
    pl.kernel,
    out_type=(
        jax.ShapeDtypeStruct((VP,), jnp.int32),
        jax.ShapeDtypeStruct((VP,), jnp.int32),
    ),
    mesh=_mesh,
    compiler_params=_params,
    scratch_types=[
        pltpu.VMEM((SHH,), jnp.int32),
        pltpu.VMEM((32, 128), jnp.int32),
        pltpu.VMEM((32, 128), jnp.int32),
        pltpu.SemaphoreType.DMA,
        pltpu.SemaphoreType.DMA,
    ],
)(_ka)


def _kb(ids_hbm, ta_hbm, tb_hbm, fpo_hbm, s_hbm, part_hbm,
        idv, fpva, fpvb, sbuf, pbuf, sem):
  wid = _wid()
  rbase = wid * CR
  pbase = rbase * 128

  pltpu.sync_copy(ids_hbm.at[pl.ds(rbase, CR), :], idv)

  def mkgrp(src, dst):
    def ggrp(g, _):
      g = g.astype(jnp.int32)
      for b in range(10):
        j = g * 10 + b
        pltpu.make_async_copy(src.at[idv.at[j]], dst.at[j], sem).start()
      for b in range(10):
        pltpu.make_async_copy(src.at[idv.at[0]], dst.at[0], sem).wait()
      return 0
    return ggrp

  lax.fori_loop(0, CR // 10, mkgrp(ta_hbm, fpva), 0)
  lax.fori_loop(0, CR // 10, mkgrp(tb_hbm, fpvb), 0)

  iot = _iota16()

  def crow(r, carry):
    r = r.astype(jnp.int32)
    for c in range(8):
      v = jnp.minimum(fpva[r, pl.ds(c * L, L)], fpvb[r, pl.ds(c * L, L)])
      fpva[r, pl.ds(c * L, L)] = v
      pos = (pbase + r * 128 + c * L) + iot
      isf = jnp.where(v == pos, 1, 0).astype(jnp.int32)
      cs = plsc.cumsum(isf)
      sbuf[pl.ds(r * 128 + c * L, L)] = cs - isf + carry
      carry = carry + jnp.sum(isf, dtype=jnp.int32)
    return carry

  total = lax.fori_loop(0, CR, crow, jnp.int32(0))

  pltpu.sync_copy(fpva, fpo_hbm.at[pl.ds(rbase, CR), :])
  pltpu.sync_copy(sbuf, s_hbm.at[pl.ds(pbase, CH)])
  pbuf[...] = jnp.full((L,), 0, jnp.int32) + total
  pltpu.sync_copy(pbuf, part_hbm.at[wid])


_phase_b = functools.partial(
    pl.kernel,
    out_type=(
        jax.ShapeDtypeStruct((ROWS, 128), jnp.int32),
        jax.ShapeDtypeStruct((N,), jnp.int32),
        jax.ShapeDtypeStruct((NW, L), jnp.int32),
    ),
    mesh=_mesh,
    compiler_params=_params,
    scratch_types=[
        pltpu.VMEM((CR, 128), jnp.int32),
        pltpu.VMEM((CR, 128), jnp.int32),
        pltpu.VMEM((CR, 128), jnp.int32),
        pltpu.VMEM((CH,), jnp.int32),
        pltpu.VMEM((L,), jnp.int32),
        pltpu.SemaphoreType.DMA,
    ],
)(_kb)


def _kc(fpo_hbm, s_hbm, part_hbm, emb_hbm, out_hbm,
        pv, offv, fpv, sv, rb0, rb1, sem, sr0, sr1):
  wid = _wid()
  rbase = wid * CR

  pltpu.sync_copy(part_hbm, pv)
  iot = _iota16()
  z = jnp.zeros((L,), jnp.int32)
  t0 = plsc.load_gather(pv, [iot, z])
  t1 = plsc.load_gather(pv, [iot + 16, z])
  c0 = plsc.cumsum(t0)
  c1 = plsc.cumsum(t1) + jnp.sum(t0, dtype=jnp.int32)
  offv[pl.ds(0, L)] = c0 - t0
  offv[pl.ds(L, L)] = c1 - t1

  pltpu.sync_copy(fpo_hbm.at[pl.ds(rbase, CR), :], fpv)

  def ggrp(g, _):
    g = g.astype(jnp.int32)
    for b in range(10):
      j = g * 10 + b
      pltpu.make_async_copy(s_hbm.at[fpv.at[j]], sv.at[j], sem).start()
    for b in range(10):
      pltpu.make_async_copy(s_hbm.at[fpv.at[0]], sv.at[0], sem).wait()
    return 0

  lax.fori_loop(0, CR // 10, ggrp, 0)

  def hrow(r, _):
    r = r.astype(jnp.int32)
    for c in range(8):
      f = fpv[r, pl.ds(c * L, L)]
      s = sv[r, pl.ds(c * L, L)]
      o = plsc.load_gather(offv, [f // CH])
      sv[r, pl.ds(c * L, L)] = s + o
    return 0

  lax.fori_loop(0, CR, hrow, 0)

  obase = wid * CH

  def gst(j, buf, sem_):
    pltpu.make_async_copy(emb_hbm.at[sv.at[j]], buf, sem_).start()

  def gwt(buf, sem_):
    pltpu.make_async_copy(emb_hbm.at[sv.at[0]], buf, sem_).wait()

  gst(0, rb0, sr0)

  def grow(g, _):
    j = 2 * g.astype(jnp.int32)
    gst(j + 1, rb1, sr1)
    gwt(rb0, sr0)
    pltpu.sync_copy(rb0, out_hbm.at[pl.ds(obase + j * 128, 128), :])

    @pl.when(j + 2 < CR)
    def _():
      gst(j + 2, rb0, sr0)

    gwt(rb1, sr1)
    pltpu.sync_copy(rb1, out_hbm.at[pl.ds(obase + (j + 1) * 128, 128), :])
    return 0

  lax.fori_loop(0, CR // 2, grow, 0)


_phase_c = functools.partial(
    pl.kernel,
    out_type=jax.ShapeDtypeStruct((N, DIM), jnp.float32),
    mesh=_mesh,
    compiler_params=_params,
    scratch_types=[
        pltpu.VMEM((NW, L), jnp.int32),
        pltpu.VMEM((NW,), jnp.int32),
        pltpu.VMEM((CR, 128), jnp.int32),
        pltpu.VMEM((CR, 128), jnp.int32),
        pltpu.VMEM((128, DIM), jnp.float32),
        pltpu.VMEM((128, DIM), jnp.float32),
        pltpu.SemaphoreType.DMA,
        pltpu.SemaphoreType.DMA,
        pltpu.SemaphoreType.DMA,
    ],
)(_kc)


def kernel(ids, embedding_var, default_embedding):
  del default_embedding
  with jax.enable_x64(False):
    ids32 = ids.astype(jnp.int32).reshape(ROWS, 128)
    emb = embedding_var[:N]
    ta, tb = _phase_a(ids32)
    fpo, s_local, partials = _phase_b(ids32, ta, tb)
    out = _phase_c(fpo, s_local, partials, emb)
  return out.reshape(ids.shape + (DIM,))


# speedup vs baseline: 150.7177x; 1.0249x over previous
import functools

import jax
import jax.numpy as jnp
from jax import lax
from jax.experimental import pallas as pl
from jax.experimental.pallas import tpu as pltpu
from jax.experimental.pallas import tpu_sc as plsc

N = 1024 * 26 * 20
DIM = 32
VOCAB = 1000000
L = 16
NC, NS = 2, 16
NW = NC * NS
CH = N // NW
ROWS = N // 128
CR = ROWS // NW
HR = ROWS // NC
N2 = N // NC
NP = HR // 32
VP = 1000448
SHH = VP // NS
SENT = 2**30

_mesh = plsc.VectorSubcoreMesh(
    core_axis_name="c", subcore_axis_name="s", num_cores=NC, num_subcores=NS)
_params = pltpu.CompilerParams(
    needs_layout_passes=False, use_tc_tiling_on_sc=False)


def _wid():
  return lax.axis_index("s") * NC + lax.axis_index("c")


def _iota16():
  return lax.iota(jnp.int32, 16)


def _ka(idr_hbm, ta_hbm, tb_hbm, tbl, ib0, ib1, s0, s1):
  sid = lax.axis_index("s")
  cid = lax.axis_index("c")
  base = sid * SHH
  hi = base + SHH

  sent = jnp.full((L,), SENT, jnp.int32)

  def init_body(i, _):
    i = i.astype(jnp.int32)
    tbl[pl.ds(i * L, L)] = sent
    return 0

  lax.fori_loop(0, SHH // L, init_body, 0, unroll=4)

  iot = _iota16()
  row0 = cid * HR

  def start(j, buf, sem):
    pltpu.make_async_copy(
        idr_hbm.at[pl.ds(row0 + j * 32, 32), :], buf, sem).start()

  def wait(buf, sem):
    pltpu.make_async_copy(idr_hbm.at[pl.ds(0, 32), :], buf, sem).wait()

  def process(buf, j):
    pbase = cid * N2 + j * 4096

    def row(rr, _):
      r = 31 - rr.astype(jnp.int32)
      for c in range(7, -1, -1):
        rid = lax.rev(buf[r, pl.ds(c * L, L)], (0,))
        m = (rid >= base) & (rid < hi)
        _, lastm = plsc.scan_count(rid, mask=m)
        lm = lastm & m
        idx = jnp.where(m, rid - base, 0)
        rpos = (pbase + r * 128 + c * L + 15) - iot
        plsc.store_scatter(tbl, [idx], rpos, mask=lm)
      return 0

    lax.fori_loop(0, 32, row, 0, unroll=2)

  start(NP - 1, ib0, s0)

  def piece(g, _):
    g = g.astype(jnp.int32)
    j0 = (NP - 1) - 2 * g

    @pl.when(j0 >= 1)
    def _():
      start(j0 - 1, ib1, s1)

    wait(ib0, s0)
    process(ib0, j0)

    j1 = j0 - 1

    @pl.when(j1 >= 0)
    def _():
      @pl.when(j1 >= 1)
      def _():
        start(j1 - 1, ib0, s0)

      wait(ib1, s1)
      process(ib1, j1)

    return 0

  lax.fori_loop(0, (NP + 1) // 2, piece, 0)

  @pl.when(cid == 0)
  def _():
    pltpu.sync_copy(tbl, ta_hbm.at[pl.ds(base, SHH)])

  @pl.when(cid == 1)
  def _():
    pltpu.sync_copy(tbl, tb_hbm.at[pl.ds(base, SHH)])


_phase_a = functools.partial(
    pl.kernel,
    out_type=(
        jax.ShapeDtypeStruct((VP,), jnp.int32),
        jax.ShapeDtypeStruct((VP,), jnp.int32),
    ),
    mesh=_mesh,
    compiler_params=_params,
    scratch_types=[
        pltpu.VMEM((SHH,), jnp.int32),
        pltpu.VMEM((32, 128), jnp.int32),
        pltpu.VMEM((32, 128), jnp.int32),
        pltpu.SemaphoreType.DMA,
        pltpu.SemaphoreType.DMA,
    ],
)(_ka)


def _kb(ids_hbm, ta_hbm, tb_hbm, fpo_hbm, s_hbm, part_hbm,
        idv, fpva, fpvb, sbuf, pbuf, sem):
  wid = _wid()
  rbase = wid * CR
  pbase = rbase * 128

  pltpu.sync_copy(ids_hbm.at[pl.ds(rbase, CR), :], idv)

  def mkgrp(src, dst):
    def ggrp(g, _):
      g = g.astype(jnp.int32)
      for b in range(10):
        j = g * 10 + b
        pltpu.make_async_copy(src.at[idv.at[j]], dst.at[j], sem).start()
      for b in range(10):
        pltpu.make_async_copy(src.at[idv.at[0]], dst.at[0], sem).wait()
      return 0
    return ggrp

  lax.fori_loop(0, CR // 10, mkgrp(ta_hbm, fpva), 0)
  lax.fori_loop(0, CR // 10, mkgrp(tb_hbm, fpvb), 0)

  iot = _iota16()

  def crow(r, carry):
    r = r.astype(jnp.int32)
    for c in range(8):
      v = jnp.minimum(fpva[r, pl.ds(c * L, L)], fpvb[r, pl.ds(c * L, L)])
      fpva[r, pl.ds(c * L, L)] = v
      pos = (pbase + r * 128 + c * L) + iot
      isf = jnp.where(v == pos, 1, 0).astype(jnp.int32)
      cs = plsc.cumsum(isf)
      sbuf[pl.ds(r * 128 + c * L, L)] = cs - isf + carry
      carry = carry + jnp.sum(isf, dtype=jnp.int32)
    return carry

  total = lax.fori_loop(0, CR, crow, jnp.int32(0))

  pltpu.sync_copy(fpva, fpo_hbm.at[pl.ds(rbase, CR), :])
  pltpu.sync_copy(sbuf, s_hbm.at[pl.ds(pbase, CH)])
  pbuf[...] = jnp.full((L,), 0, jnp.int32) + total
  pltpu.sync_copy(pbuf, part_hbm.at[wid])


_phase_b = functools.partial(
    pl.kernel,
    out_type=(
        jax.ShapeDtypeStruct((ROWS, 128), jnp.int32),
        jax.ShapeDtypeStruct((N,), jnp.int32),
        jax.ShapeDtypeStruct((NW, L), jnp.int32),
    ),
    mesh=_mesh,
    compiler_params=_params,
    scratch_types=[
        pltpu.VMEM((CR, 128), jnp.int32),
        pltpu.VMEM((CR, 128), jnp.int32),
        pltpu.VMEM((CR, 128), jnp.int32),
        pltpu.VMEM((CH,), jnp.int32),
        pltpu.VMEM((L,), jnp.int32),
        pltpu.SemaphoreType.DMA,
    ],
)(_kb)


def _kc(fpo_hbm, s_hbm, part_hbm, emb_hbm, out_hbm,
        pv, offv, fpv, sv, rb0, rb1, rb2, rb3,
        sem, sg0, sg1, sg2, sg3, sw0, sw1, sw2, sw3):
  wid = _wid()
  rbase = wid * CR

  pltpu.sync_copy(part_hbm, pv)
  iot = _iota16()
  z = jnp.zeros((L,), jnp.int32)
  t0 = plsc.load_gather(pv, [iot, z])
  t1 = plsc.load_gather(pv, [iot + 16, z])
  c0 = plsc.cumsum(t0)
  c1 = plsc.cumsum(t1) + jnp.sum(t0, dtype=jnp.int32)
  offv[pl.ds(0, L)] = c0 - t0
  offv[pl.ds(L, L)] = c1 - t1

  pltpu.sync_copy(fpo_hbm.at[pl.ds(rbase, CR), :], fpv)

  def ggrp(g, _):
    g = g.astype(jnp.int32)
    for b in range(10):
      j = g * 10 + b
      pltpu.make_async_copy(s_hbm.at[fpv.at[j]], sv.at[j], sem).start()
    for b in range(10):
      pltpu.make_async_copy(s_hbm.at[fpv.at[0]], sv.at[0], sem).wait()
    return 0

  lax.fori_loop(0, CR // 10, ggrp, 0)

  def hrow(r, _):
    r = r.astype(jnp.int32)
    for c in range(8):
      f = fpv[r, pl.ds(c * L, L)]
      s = sv[r, pl.ds(c * L, L)]
      o = plsc.load_gather(offv, [f // CH])
      sv[r, pl.ds(c * L, L)] = s + o
    return 0

  lax.fori_loop(0, CR, hrow, 0)

  obase = wid * CH
  rbs = (rb0, rb1, rb2, rb3)
  sgs = (sg0, sg1, sg2, sg3)
  sws = (sw0, sw1, sw2, sw3)

  def gst(j, b):
    pltpu.make_async_copy(emb_hbm.at[sv.at[j]], rbs[b], sgs[b]).start()

  def gwt(b):
    pltpu.make_async_copy(emb_hbm.at[sv.at[0]], rbs[b], sgs[b]).wait()

  def wst(j, b):
    pltpu.make_async_copy(
        rbs[b], out_hbm.at[pl.ds(obase + j * 128, 128), :], sws[b]).start()

  def wwt(b):
    pltpu.make_async_copy(
        rbs[b], out_hbm.at[pl.ds(obase, 128), :], sws[b]).wait()

  for b in range(4):
    gst(b, b)

  def grow(g, _):
    j0 = 4 * g.astype(jnp.int32)
    for b in range(4):
      j = j0 + b

      @pl.when(j < CR)
      def _():
        gwt(b)
        wst(j, b)

    for b in range(4):
      jn = j0 + 4 + b

      @pl.when(jn < CR)
      def _():
        wwt(b)
        gst(jn, b)

    return 0

  lax.fori_loop(0, (CR + 3) // 4, grow, 0)

  for b in range(4):
    wwt(b)


_phase_c = functools.partial(
    pl.kernel,
    out_type=jax.ShapeDtypeStruct((N, DIM), jnp.float32),
    mesh=_mesh,
    compiler_params=_params,
    scratch_types=[
        pltpu.VMEM((NW, L), jnp.int32),
        pltpu.VMEM((NW,), jnp.int32),
        pltpu.VMEM((CR, 128), jnp.int32),
        pltpu.VMEM((CR, 128), jnp.int32),
        pltpu.VMEM((128, DIM), jnp.float32),
        pltpu.VMEM((128, DIM), jnp.float32),
        pltpu.VMEM((128, DIM), jnp.float32),
        pltpu.VMEM((128, DIM), jnp.float32),
        pltpu.SemaphoreType.DMA,
        pltpu.SemaphoreType.DMA,
        pltpu.SemaphoreType.DMA,
        pltpu.SemaphoreType.DMA,
        pltpu.SemaphoreType.DMA,
        pltpu.SemaphoreType.DMA,
        pltpu.SemaphoreType.DMA,
        pltpu.SemaphoreType.DMA,
        pltpu.SemaphoreType.DMA,
    ],
)(_kc)


def kernel(ids, embedding_var, default_embedding):
  del default_embedding
  with jax.enable_x64(False):
    ids32 = ids.astype(jnp.int32).reshape(ROWS, 128)
    emb = embedding_var[:N]
    ta, tb = _phase_a(ids32)
    fpo, s_local, partials = _phase_b(ids32, ta, tb)
    out = _phase_c(fpo, s_local, partials, emb)
  return out.reshape(ids.shape + (DIM,))


# speedup vs baseline: 190.5673x; 1.2644x over previous
import functools

import jax
import jax.numpy as jnp
from jax import lax
from jax.experimental import pallas as pl
from jax.experimental.pallas import tpu as pltpu
from jax.experimental.pallas import tpu_sc as plsc

N = 1024 * 26 * 20
DIM = 32
VOCAB = 1000000
L = 16
NC, NS = 2, 16
NW = NC * NS
CH = N // NW
ROWS = N // 128
CR = ROWS // NW
HR = ROWS // NC
N2 = N // NC
NP = HR // 32
VP = 1000448
SHH = VP // NS
SENT = 2**30

_mesh = plsc.VectorSubcoreMesh(
    core_axis_name="c", subcore_axis_name="s", num_cores=NC, num_subcores=NS)
_params = pltpu.CompilerParams(
    needs_layout_passes=False, use_tc_tiling_on_sc=False)


def _wid():
  return lax.axis_index("s") * NC + lax.axis_index("c")


def _iota16():
  return lax.iota(jnp.int32, 16)


def _ka(idr_hbm, ta_hbm, tb_hbm, tbl, ib0, ib1, s0, s1):
  sid = lax.axis_index("s")
  cid = lax.axis_index("c")
  base = sid * SHH
  hi = base + SHH

  sent = jnp.full((L,), SENT, jnp.int32)

  def init_body(i, _):
    i = i.astype(jnp.int32)
    tbl[pl.ds(i * L, L)] = sent
    return 0

  lax.fori_loop(0, SHH // L, init_body, 0, unroll=4)

  iot = _iota16()
  row0 = cid * HR

  def start(j, buf, sem):
    pltpu.make_async_copy(
        idr_hbm.at[pl.ds(row0 + j * 32, 32), :], buf, sem).start()

  def wait(buf, sem):
    pltpu.make_async_copy(idr_hbm.at[pl.ds(0, 32), :], buf, sem).wait()

  def process(buf, j):
    pbase = cid * N2 + j * 4096

    def row(rr, _):
      r = 31 - rr.astype(jnp.int32)
      lms, idxs, rps = [], [], []
      for c in range(7, -1, -1):
        rid = lax.rev(buf[r, pl.ds(c * L, L)], (0,))
        m = (rid >= base) & (rid < hi)
        _, lastm = plsc.scan_count(rid, mask=m)
        lms.append(lastm & m)
        idxs.append(jnp.where(m, rid - base, 0))
        rps.append((pbase + r * 128 + c * L + 15) - iot)
      for k in range(8):
        plsc.store_scatter(tbl, [idxs[k]], rps[k], mask=lms[k])
      return 0

    lax.fori_loop(0, 32, row, 0, unroll=2)

  start(NP - 1, ib0, s0)

  def piece(g, _):
    g = g.astype(jnp.int32)
    j0 = (NP - 1) - 2 * g

    @pl.when(j0 >= 1)
    def _():
      start(j0 - 1, ib1, s1)

    wait(ib0, s0)
    process(ib0, j0)

    j1 = j0 - 1

    @pl.when(j1 >= 0)
    def _():
      @pl.when(j1 >= 1)
      def _():
        start(j1 - 1, ib0, s0)

      wait(ib1, s1)
      process(ib1, j1)

    return 0

  lax.fori_loop(0, (NP + 1) // 2, piece, 0)

  @pl.when(cid == 0)
  def _():
    pltpu.sync_copy(tbl, ta_hbm.at[pl.ds(base, SHH)])

  @pl.when(cid == 1)
  def _():
    pltpu.sync_copy(tbl, tb_hbm.at[pl.ds(base, SHH)])


_phase_a = functools.partial(
    pl.kernel,
    out_type=(
        jax.ShapeDtypeStruct((VP,), jnp.int32),
        jax.ShapeDtypeStruct((VP,), jnp.int32),
    ),
    mesh=_mesh,
    compiler_params=_params,
    scratch_types=[
        pltpu.VMEM((SHH,), jnp.int32),
        pltpu.VMEM((32, 128), jnp.int32),
        pltpu.VMEM((32, 128), jnp.int32),
        pltpu.SemaphoreType.DMA,
        pltpu.SemaphoreType.DMA,
    ],
)(_ka)


def _kb(ids_hbm, ta_hbm, tb_hbm, fpo_hbm, s_hbm, part_hbm,
        idv, fpva, fpvb, sbuf, pbuf, sa, sb):
  wid = _wid()
  rbase = wid * CR
  pbase = rbase * 128

  pltpu.sync_copy(ids_hbm.at[pl.ds(rbase, CR), :], idv)

  iot = _iota16()
  GR = 5

  def issue(g):
    for b in range(GR):
      j = g * GR + b
      pltpu.make_async_copy(ta_hbm.at[idv.at[j]], fpva.at[j], sa).start()
      pltpu.make_async_copy(tb_hbm.at[idv.at[j]], fpvb.at[j], sb).start()

  def drain():
    for b in range(GR):
      pltpu.make_async_copy(ta_hbm.at[idv.at[0]], fpva.at[0], sa).wait()
      pltpu.make_async_copy(tb_hbm.at[idv.at[0]], fpvb.at[0], sb).wait()

  def crow(r, carry):
    for c in range(8):
      v = jnp.minimum(fpva[r, pl.ds(c * L, L)], fpvb[r, pl.ds(c * L, L)])
      fpva[r, pl.ds(c * L, L)] = v
      pos = (pbase + r * 128 + c * L) + iot
      isf = jnp.where(v == pos, 1, 0).astype(jnp.int32)
      cs = plsc.cumsum(isf)
      sbuf[pl.ds(r * 128 + c * L, L)] = cs - isf + carry
      carry = carry + jnp.sum(isf, dtype=jnp.int32)
    return carry

  issue(jnp.int32(0))

  def grp(g, carry):
    g = g.astype(jnp.int32)

    @pl.when(g + 1 < CR // GR)
    def _():
      issue(g + 1)

    drain()
    for b in range(GR):
      carry = crow(g * GR + b, carry)
    return carry

  total = lax.fori_loop(0, CR // GR, grp, jnp.int32(0))

  pltpu.sync_copy(fpva, fpo_hbm.at[pl.ds(rbase, CR), :])
  pltpu.sync_copy(sbuf, s_hbm.at[pl.ds(pbase, CH)])
  pbuf[...] = jnp.full((L,), 0, jnp.int32) + total
  pltpu.sync_copy(pbuf, part_hbm.at[wid])


_phase_b = functools.partial(
    pl.kernel,
    out_type=(
        jax.ShapeDtypeStruct((ROWS, 128), jnp.int32),
        jax.ShapeDtypeStruct((N,), jnp.int32),
        jax.ShapeDtypeStruct((NW, L), jnp.int32),
    ),
    mesh=_mesh,
    compiler_params=_params,
    scratch_types=[
        pltpu.VMEM((CR, 128), jnp.int32),
        pltpu.VMEM((CR, 128), jnp.int32),
        pltpu.VMEM((CR, 128), jnp.int32),
        pltpu.VMEM((CH,), jnp.int32),
        pltpu.VMEM((L,), jnp.int32),
        pltpu.SemaphoreType.DMA,
        pltpu.SemaphoreType.DMA,
    ],
)(_kb)


def _kc(fpo_hbm, s_hbm, part_hbm, emb_hbm, out_hbm,
        pv, offv, fpv, sv, rb0, rb1, rb2, rb3,
        sem, sg0, sg1, sg2, sg3, sw0, sw1, sw2, sw3):
  wid = _wid()
  rbase = wid * CR

  pltpu.sync_copy(part_hbm, pv)
  iot = _iota16()
  z = jnp.zeros((L,), jnp.int32)
  t0 = plsc.load_gather(pv, [iot, z])
  t1 = plsc.load_gather(pv, [iot + 16, z])
  c0 = plsc.cumsum(t0)
  c1 = plsc.cumsum(t1) + jnp.sum(t0, dtype=jnp.int32)
  offv[pl.ds(0, L)] = c0 - t0
  offv[pl.ds(L, L)] = c1 - t1

  pltpu.sync_copy(fpo_hbm.at[pl.ds(rbase, CR), :], fpv)

  def ggrp(g, _):
    g = g.astype(jnp.int32)
    for b in range(10):
      j = g * 10 + b
      pltpu.make_async_copy(s_hbm.at[fpv.at[j]], sv.at[j], sem).start()
    for b in range(10):
      pltpu.make_async_copy(s_hbm.at[fpv.at[0]], sv.at[0], sem).wait()
    return 0

  lax.fori_loop(0, CR // 10, ggrp, 0)

  def hrow(r, _):
    r = r.astype(jnp.int32)
    for c in range(8):
      f = fpv[r, pl.ds(c * L, L)]
      s = sv[r, pl.ds(c * L, L)]
      o = plsc.load_gather(offv, [f // CH])
      sv[r, pl.ds(c * L, L)] = s + o
    return 0

  lax.fori_loop(0, CR, hrow, 0)

  obase = wid * CH
  rbs = (rb0, rb1, rb2, rb3)
  sgs = (sg0, sg1, sg2, sg3)
  sws = (sw0, sw1, sw2, sw3)

  def gst(j, b):
    pltpu.make_async_copy(emb_hbm.at[sv.at[j]], rbs[b], sgs[b]).start()

  def gwt(b):
    pltpu.make_async_copy(emb_hbm.at[sv.at[0]], rbs[b], sgs[b]).wait()

  def wst(j, b):
    pltpu.make_async_copy(
        rbs[b], out_hbm.at[pl.ds(obase + j * 128, 128), :], sws[b]).start()

  def wwt(b):
    pltpu.make_async_copy(
        rbs[b], out_hbm.at[pl.ds(obase, 128), :], sws[b]).wait()

  for b in range(4):
    gst(b, b)

  def grow(g, _):
    j0 = 4 * g.astype(jnp.int32)
    for b in range(4):
      j = j0 + b

      @pl.when(j < CR)
      def _():
        gwt(b)
        wst(j, b)

    for b in range(4):
      jn = j0 + 4 + b

      @pl.when(jn < CR)
      def _():
        wwt(b)
        gst(jn, b)

    return 0

  lax.fori_loop(0, (CR + 3) // 4, grow, 0)

  for b in range(4):
    wwt(b)


_phase_c = functools.partial(
    pl.kernel,
    out_type=jax.ShapeDtypeStruct((N, DIM), jnp.float32),
    mesh=_mesh,
    compiler_params=_params,
    scratch_types=[
        pltpu.VMEM((NW, L), jnp.int32),
        pltpu.VMEM((NW,), jnp.int32),
        pltpu.VMEM((CR, 128), jnp.int32),
        pltpu.VMEM((CR, 128), jnp.int32),
        pltpu.VMEM((128, DIM), jnp.float32),
        pltpu.VMEM((128, DIM), jnp.float32),
        pltpu.VMEM((128, DIM), jnp.float32),
        pltpu.VMEM((128, DIM), jnp.float32),
        pltpu.SemaphoreType.DMA,
        pltpu.SemaphoreType.DMA,
        pltpu.SemaphoreType.DMA,
        pltpu.SemaphoreType.DMA,
        pltpu.SemaphoreType.DMA,
        pltpu.SemaphoreType.DMA,
        pltpu.SemaphoreType.DMA,
        pltpu.SemaphoreType.DMA,
        pltpu.SemaphoreType.DMA,
    ],
)(_kc)


def kernel(ids, embedding_var, default_embedding):
  del default_embedding
  with jax.enable_x64(False):
    ids32 = ids.astype(jnp.int32).reshape(ROWS, 128)
    emb = embedding_var[:N]
    ta, tb = _phase_a(ids32)
    fpo, s_local, partials = _phase_b(ids32, ta, tb)
    out = _phase_c(fpo, s_local, partials, emb)
  return out.reshape(ids.shape + (DIM,))


# speedup vs baseline: 190.6114x; 1.0002x over previous
import functools

import jax
import jax.numpy as jnp
from jax import lax
from jax.experimental import pallas as pl
from jax.experimental.pallas import tpu as pltpu
from jax.experimental.pallas import tpu_sc as plsc

N = 1024 * 26 * 20
DIM = 32
VOCAB = 1000000
L = 16
NC, NS = 2, 16
NW = NC * NS
CH = N // NW
ROWS = N // 128
CR = ROWS // NW
HR = ROWS // NC
N2 = N // NC
NP = HR // 32
VP = 1000448
SHH = VP // NS
SENT = 2**30

_mesh = plsc.VectorSubcoreMesh(
    core_axis_name="c", subcore_axis_name="s", num_cores=NC, num_subcores=NS)
_params = pltpu.CompilerParams(
    needs_layout_passes=False, use_tc_tiling_on_sc=False)


def _wid():
  return lax.axis_index("s") * NC + lax.axis_index("c")


def _iota16():
  return lax.iota(jnp.int32, 16)


def _ka(idr_hbm, ta_hbm, tb_hbm, tbl, ib0, ib1, s0, s1):
  sid = lax.axis_index("s")
  cid = lax.axis_index("c")
  base = sid * SHH
  hi = base + SHH

  sent = jnp.full((L,), SENT, jnp.int32)

  def init_body(i, _):
    i = i.astype(jnp.int32)
    tbl[pl.ds(i * L, L)] = sent
    return 0

  lax.fori_loop(0, SHH // L, init_body, 0, unroll=4)

  iot = _iota16()
  row0 = cid * HR

  def start(j, buf, sem):
    pltpu.make_async_copy(
        idr_hbm.at[pl.ds(row0 + j * 32, 32), :], buf, sem).start()

  def wait(buf, sem):
    pltpu.make_async_copy(idr_hbm.at[pl.ds(0, 32), :], buf, sem).wait()

  def process(buf, j):
    pbase = cid * N2 + j * 4096

    def row(rr, _):
      r = 31 - rr.astype(jnp.int32)
      lms, idxs, rps = [], [], []
      for c in range(7, -1, -1):
        rid = lax.rev(buf[r, pl.ds(c * L, L)], (0,))
        m = (rid >= base) & (rid < hi)
        _, lastm = plsc.scan_count(rid, mask=m)
        lms.append(lastm & m)
        idxs.append(jnp.where(m, rid - base, 0))
        rps.append((pbase + r * 128 + c * L + 15) - iot)
      for k in range(8):
        plsc.store_scatter(tbl, [idxs[k]], rps[k], mask=lms[k])
      return 0

    lax.fori_loop(0, 32, row, 0, unroll=2)

  start(NP - 1, ib0, s0)

  def piece(g, _):
    g = g.astype(jnp.int32)
    j0 = (NP - 1) - 2 * g

    @pl.when(j0 >= 1)
    def _():
      start(j0 - 1, ib1, s1)

    wait(ib0, s0)
    process(ib0, j0)

    j1 = j0 - 1

    @pl.when(j1 >= 0)
    def _():
      @pl.when(j1 >= 1)
      def _():
        start(j1 - 1, ib0, s0)

      wait(ib1, s1)
      process(ib1, j1)

    return 0

  lax.fori_loop(0, (NP + 1) // 2, piece, 0)

  @pl.when(cid == 0)
  def _():
    pltpu.sync_copy(tbl, ta_hbm.at[pl.ds(base, SHH)])

  @pl.when(cid == 1)
  def _():
    pltpu.sync_copy(tbl, tb_hbm.at[pl.ds(base, SHH)])


_phase_a = functools.partial(
    pl.kernel,
    out_type=(
        jax.ShapeDtypeStruct((VP,), jnp.int32),
        jax.ShapeDtypeStruct((VP,), jnp.int32),
    ),
    mesh=_mesh,
    compiler_params=_params,
    scratch_types=[
        pltpu.VMEM((SHH,), jnp.int32),
        pltpu.VMEM((32, 128), jnp.int32),
        pltpu.VMEM((32, 128), jnp.int32),
        pltpu.SemaphoreType.DMA,
        pltpu.SemaphoreType.DMA,
    ],
)(_ka)


def _kb(ids_hbm, ta_hbm, tb_hbm, fpo_hbm, s_hbm, part_hbm,
        idv, fpva, fpvb, sbuf, pbuf, sa, sb):
  wid = _wid()
  rbase = wid * CR
  pbase = rbase * 128

  pltpu.sync_copy(ids_hbm.at[pl.ds(rbase, CR), :], idv)

  iot = _iota16()
  GR = 5

  def issue(g):
    for b in range(GR):
      j = g * GR + b
      pltpu.make_async_copy(ta_hbm.at[idv.at[j]], fpva.at[j], sa).start()
      pltpu.make_async_copy(tb_hbm.at[idv.at[j]], fpvb.at[j], sb).start()

  def drain():
    for b in range(GR):
      pltpu.make_async_copy(ta_hbm.at[idv.at[0]], fpva.at[0], sa).wait()
      pltpu.make_async_copy(tb_hbm.at[idv.at[0]], fpvb.at[0], sb).wait()

  def crow(r, carry):
    exs, tots = [], []
    for c in range(8):
      v = jnp.minimum(fpva[r, pl.ds(c * L, L)], fpvb[r, pl.ds(c * L, L)])
      fpva[r, pl.ds(c * L, L)] = v
      pos = (pbase + r * 128 + c * L) + iot
      isf = jnp.where(v == pos, 1, 0).astype(jnp.int32)
      exs.append(plsc.cumsum(isf) - isf)
      tots.append(jnp.sum(isf, dtype=jnp.int32))
    for c in range(8):
      sbuf[pl.ds(r * 128 + c * L, L)] = exs[c] + carry
      carry = carry + tots[c]
    return carry

  issue(jnp.int32(0))

  def grp(g, carry):
    g = g.astype(jnp.int32)

    @pl.when(g + 1 < CR // GR)
    def _():
      issue(g + 1)

    drain()
    for b in range(GR):
      carry = crow(g * GR + b, carry)
    return carry

  total = lax.fori_loop(0, CR // GR, grp, jnp.int32(0))

  pltpu.sync_copy(fpva, fpo_hbm.at[pl.ds(rbase, CR), :])
  pltpu.sync_copy(sbuf, s_hbm.at[pl.ds(pbase, CH)])
  pbuf[...] = jnp.full((L,), 0, jnp.int32) + total
  pltpu.sync_copy(pbuf, part_hbm.at[wid])


_phase_b = functools.partial(
    pl.kernel,
    out_type=(
        jax.ShapeDtypeStruct((ROWS, 128), jnp.int32),
        jax.ShapeDtypeStruct((N,), jnp.int32),
        jax.ShapeDtypeStruct((NW, L), jnp.int32),
    ),
    mesh=_mesh,
    compiler_params=_params,
    scratch_types=[
        pltpu.VMEM((CR, 128), jnp.int32),
        pltpu.VMEM((CR, 128), jnp.int32),
        pltpu.VMEM((CR, 128), jnp.int32),
        pltpu.VMEM((CH,), jnp.int32),
        pltpu.VMEM((L,), jnp.int32),
        pltpu.SemaphoreType.DMA,
        pltpu.SemaphoreType.DMA,
    ],
)(_kb)


def _kc(fpo_hbm, s_hbm, part_hbm, emb_hbm, out_hbm,
        pv, offv, fpv, sv, rb0, rb1, rb2, rb3,
        sem, sg0, sg1, sg2, sg3, sw0, sw1, sw2, sw3):
  wid = _wid()
  rbase = wid * CR

  pltpu.sync_copy(part_hbm, pv)
  iot = _iota16()
  z = jnp.zeros((L,), jnp.int32)
  t0 = plsc.load_gather(pv, [iot, z])
  t1 = plsc.load_gather(pv, [iot + 16, z])
  c0 = plsc.cumsum(t0)
  c1 = plsc.cumsum(t1) + jnp.sum(t0, dtype=jnp.int32)
  offv[pl.ds(0, L)] = c0 - t0
  offv[pl.ds(L, L)] = c1 - t1

  pltpu.sync_copy(fpo_hbm.at[pl.ds(rbase, CR), :], fpv)

  def ggrp(g, _):
    g = g.astype(jnp.int32)
    for b in range(10):
      j = g * 10 + b
      pltpu.make_async_copy(s_hbm.at[fpv.at[j]], sv.at[j], sem).start()
    for b in range(10):
      pltpu.make_async_copy(s_hbm.at[fpv.at[0]], sv.at[0], sem).wait()
    return 0

  lax.fori_loop(0, CR // 10, ggrp, 0)

  def hrow(r, _):
    r = r.astype(jnp.int32)
    for c in range(8):
      f = fpv[r, pl.ds(c * L, L)]
      s = sv[r, pl.ds(c * L, L)]
      o = plsc.load_gather(offv, [f // CH])
      sv[r, pl.ds(c * L, L)] = s + o
    return 0

  lax.fori_loop(0, CR, hrow, 0)

  obase = wid * CH
  rbs = (rb0, rb1, rb2, rb3)
  sgs = (sg0, sg1, sg2, sg3)
  sws = (sw0, sw1, sw2, sw3)

  def gst(j, b):
    pltpu.make_async_copy(emb_hbm.at[sv.at[j]], rbs[b], sgs[b]).start()

  def gwt(b):
    pltpu.make_async_copy(emb_hbm.at[sv.at[0]], rbs[b], sgs[b]).wait()

  def wst(j, b):
    pltpu.make_async_copy(
        rbs[b], out_hbm.at[pl.ds(obase + j * 128, 128), :], sws[b]).start()

  def wwt(b):
    pltpu.make_async_copy(
        rbs[b], out_hbm.at[pl.ds(obase, 128), :], sws[b]).wait()

  for b in range(4):
    gst(b, b)

  def grow(g, _):
    j0 = 4 * g.astype(jnp.int32)
    for b in range(4):
      j = j0 + b

      @pl.when(j < CR)
      def _():
        gwt(b)
        wst(j, b)

    for b in range(4):
      jn = j0 + 4 + b

      @pl.when(jn < CR)
      def _():
        wwt(b)
        gst(jn, b)

    return 0

  lax.fori_loop(0, (CR + 3) // 4, grow, 0)

  for b in range(4):
    wwt(b)


_phase_c = functools.partial(
    pl.kernel,
    out_type=jax.ShapeDtypeStruct((N, DIM), jnp.float32),
    mesh=_mesh,
    compiler_params=_params,
    scratch_types=[
        pltpu.VMEM((NW, L), jnp.int32),
        pltpu.VMEM((NW,), jnp.int32),
        pltpu.VMEM((CR, 128), jnp.int32),
        pltpu.VMEM((CR, 128), jnp.int32),
        pltpu.VMEM((128, DIM), jnp.float32),
        pltpu.VMEM((128, DIM), jnp.float32),
        pltpu.VMEM((128, DIM), jnp.float32),
        pltpu.VMEM((128, DIM), jnp.float32),
        pltpu.SemaphoreType.DMA,
        pltpu.SemaphoreType.DMA,
        pltpu.SemaphoreType.DMA,
        pltpu.SemaphoreType.DMA,
        pltpu.SemaphoreType.DMA,
        pltpu.SemaphoreType.DMA,
        pltpu.SemaphoreType.DMA,
        pltpu.SemaphoreType.DMA,
        pltpu.SemaphoreType.DMA,
    ],
)(_kc)


def kernel(ids, embedding_var, default_embedding):
  del default_embedding
  with jax.enable_x64(False):
    ids32 = ids.astype(jnp.int32).reshape(ROWS, 128)
    emb = embedding_var[:N]
    ta, tb = _phase_a(ids32)
    fpo, s_local, partials = _phase_b(ids32, ta, tb)
    out = _phase_c(fpo, s_local, partials, emb)
  return out.reshape(ids.shape + (DIM,))
